# baseline (speedup 1.0000x reference)
import functools

import jax
import jax.numpy as jnp
from jax import lax
from jax.experimental import pallas as pl
from jax.experimental.pallas import tpu as pltpu
from jax.experimental.pallas import tpu_sc as plsc

D = 64
SCALE = 8.0

NC = 2
NS = 16
NW = NC * NS

BATCH = 4096
SEQ = 200
BW = BATCH // NW
TT = SEQ // 8
NBUF = 2
L = 16


@functools.partial(
    pl.kernel,
    out_type=jax.ShapeDtypeStruct((SEQ, D // 8, NW, 8 * BW), jnp.float32),
    mesh=plsc.VectorSubcoreMesh(core_axis_name="c", subcore_axis_name="s"),
    compiler_params=pltpu.CompilerParams(
        use_tc_tiling_on_sc=False, needs_layout_passes=False),
    scratch_types=[
        pltpu.VMEM((TT, 8, BW), jnp.int32),
        pltpu.VMEM((BW, D), jnp.float32),
        pltpu.VMEM((BW, D), jnp.float32),
        pltpu.VMEM((BW, D), jnp.float32),
        pltpu.VMEM((BW, D), jnp.float32),
        pltpu.VMEM((D * BW,), jnp.float32),
        pltpu.VMEM((D * BW,), jnp.float32),
        pltpu.VMEM((D * BW,), jnp.float32),
        pltpu.VMEM((D * BW,), jnp.float32),
        pltpu.SemaphoreType.DMA,
        pltpu.SemaphoreType.DMA,
        pltpu.SemaphoreType.DMA,
        pltpu.SemaphoreType.DMA,
        pltpu.SemaphoreType.DMA,
        pltpu.SemaphoreType.DMA,
        pltpu.SemaphoreType.DMA,
        pltpu.SemaphoreType.DMA,
    ],
)
def _sc_embed(tok_hbm, table_hbm, out_hbm,
              idx_v, s0, s1, s2, s3, t0, t1, t2, t3,
              g0, g1, g2, g3, w0, w1, w2, w3):
    slab = (s0, s1, s2, s3)
    slabt = (t0, t1, t2, t3)
    gsem = (g0, g1, g2, g3)
    wsem = (w0, w1, w2, w3)

    wid = lax.axis_index("s") * NC + lax.axis_index("c")

    pltpu.sync_copy(tok_hbm.at[pl.ds(0, TT), wid], idx_v)

    lane = lax.iota(jnp.int32, L)
    diag = [jnp.bitwise_and(lane + s, L - 1) for s in range(L)]

    def fire(r, b):
        pltpu.async_copy(
            table_hbm.at[idx_v.at[r // 8, r % 8]], slab[b], gsem[b])

    def drain(r, b):
        pltpu.make_async_copy(
            table_hbm.at[idx_v.at[r // 8, r % 8]], slab[b], gsem[b]).wait()

    def write(r, b):
        for dt in range(D // 8):
            pltpu.async_copy(
                slabt[b].at[pl.ds(dt * 8 * BW, 8 * BW)],
                out_hbm.at[r, dt, wid], wsem[b])

    def wait_write(r, b):
        for dt in range(D // 8):
            pltpu.make_async_copy(
                slabt[b].at[pl.ds(dt * 8 * BW, 8 * BW)],
                out_hbm.at[r, dt, wid], wsem[b]).wait()

    def transpose_scale(b):
        src, dst = slab[b], slabt[b]

        @plsc.parallel_loop(0, BW // L, unroll=2)
        def _(bg):
            bvec = bg * L + lane
            for dg in range(D // L):
                for s in range(L):
                    dvec = dg * L + diag[s]
                    v = plsc.load_gather(src, [bvec, dvec])
                    plsc.store_scatter(dst, [dvec * BW + bvec], v * SCALE)

    for r in range(NBUF - 1):
        fire(r, r)

    def step(p, carry):
        for b in range(NBUF):
            r = p * NBUF + b
            drain(r, b)

            @pl.when(r >= NBUF)
            def _():
                wait_write(r - NBUF, b)

            transpose_scale(b)
            write(r, b)
            fr = r + NBUF - 1

            @pl.when(fr < SEQ)
            def _():
                fire(fr, (b + NBUF - 1) % NBUF)
        return carry

    lax.fori_loop(0, SEQ // NBUF, step, 0)

    for b in range(NBUF):
        wait_write(SEQ - NBUF + b, b)


def kernel(tokens, table):
    tok4 = jnp.transpose(
        jnp.transpose(jnp.asarray(tokens, jnp.int32)).reshape(TT, 8, NW, BW),
        (0, 2, 1, 3))
    out5 = _sc_embed(tok4, jnp.asarray(table, jnp.float32))
    return (out5.reshape(SEQ, D // 8, NW, 8, BW)
            .transpose(2, 4, 0, 1, 3)
            .reshape(BATCH, SEQ, D))


# speedup vs baseline: 1.7824x; 1.7824x over previous
import functools

import jax
import jax.numpy as jnp
from jax import lax
from jax.experimental import pallas as pl
from jax.experimental.pallas import tpu as pltpu
from jax.experimental.pallas import tpu_sc as plsc

D = 64
SCALE = 8.0

NC = 2
NS = 16
NW = NC * NS

BATCH = 4096
SEQ = 200
BW = BATCH // NW
TT = SEQ // 8
NBUF = 4
L = 16


@functools.partial(
    pl.kernel,
    out_type=jax.ShapeDtypeStruct((SEQ, D // 8, NW, 8 * BW), jnp.float32),
    mesh=plsc.VectorSubcoreMesh(core_axis_name="c", subcore_axis_name="s"),
    compiler_params=pltpu.CompilerParams(
        use_tc_tiling_on_sc=False, needs_layout_passes=False),
    scratch_types=[
        pltpu.VMEM((TT, 8, BW), jnp.int32),
        pltpu.VMEM((BW, D), jnp.float32),
        pltpu.VMEM((BW, D), jnp.float32),
        pltpu.VMEM((BW, D), jnp.float32),
        pltpu.VMEM((BW, D), jnp.float32),
        pltpu.VMEM((D * BW,), jnp.float32),
        pltpu.VMEM((D * BW,), jnp.float32),
        pltpu.VMEM((D * BW,), jnp.float32),
        pltpu.VMEM((D * BW,), jnp.float32),
        pltpu.SemaphoreType.DMA,
        pltpu.SemaphoreType.DMA,
        pltpu.SemaphoreType.DMA,
        pltpu.SemaphoreType.DMA,
        pltpu.SemaphoreType.DMA,
        pltpu.SemaphoreType.DMA,
        pltpu.SemaphoreType.DMA,
        pltpu.SemaphoreType.DMA,
    ],
)
def _sc_embed(tok_hbm, table_hbm, out_hbm,
              idx_v, s0, s1, s2, s3, t0, t1, t2, t3,
              g0, g1, g2, g3, w0, w1, w2, w3):
    slab = (s0, s1, s2, s3)
    slabt = (t0, t1, t2, t3)
    gsem = (g0, g1, g2, g3)
    wsem = (w0, w1, w2, w3)

    wid = lax.axis_index("s") * NC + lax.axis_index("c")

    pltpu.sync_copy(tok_hbm.at[pl.ds(0, TT), wid], idx_v)

    lane = lax.iota(jnp.int32, L)
    diag = [jnp.bitwise_and(lane + s, L - 1) for s in range(L)]

    def fire(r, b):
        pltpu.async_copy(
            table_hbm.at[idx_v.at[r // 8, r % 8]], slab[b], gsem[b])

    def drain(r, b):
        pltpu.make_async_copy(
            table_hbm.at[idx_v.at[r // 8, r % 8]], slab[b], gsem[b]).wait()

    def write(r, b):
        for dt in range(D // 8):
            pltpu.async_copy(
                slabt[b].at[pl.ds(dt * 8 * BW, 8 * BW)],
                out_hbm.at[r, dt, wid], wsem[b])

    def wait_write(r, b):
        for dt in range(D // 8):
            pltpu.make_async_copy(
                slabt[b].at[pl.ds(dt * 8 * BW, 8 * BW)],
                out_hbm.at[r, dt, wid], wsem[b]).wait()

    def transpose_scale(b):
        src, dst = slab[b], slabt[b]

        @plsc.parallel_loop(0, BW // L, unroll=1)
        def _(bg):
            bvec = bg * L + lane
            for dg in range(D // L):
                for s in range(L):
                    dvec = dg * L + diag[s]
                    v = plsc.load_gather(src, [bvec, dvec])
                    plsc.store_scatter(dst, [dvec * BW + bvec], v * SCALE)

    for r in range(NBUF - 1):
        fire(r, r)

    def step(p, carry):
        for b in range(NBUF):
            r = p * NBUF + b
            drain(r, b)

            @pl.when(r >= NBUF)
            def _():
                wait_write(r - NBUF, b)

            transpose_scale(b)
            write(r, b)
            fr = r + NBUF - 1

            @pl.when(fr < SEQ)
            def _():
                fire(fr, (b + NBUF - 1) % NBUF)
        return carry

    lax.fori_loop(0, SEQ // NBUF, step, 0)

    for b in range(NBUF):
        wait_write(SEQ - NBUF + b, b)


def kernel(tokens, table):
    tok4 = jnp.transpose(
        jnp.transpose(jnp.asarray(tokens, jnp.int32)).reshape(TT, 8, NW, BW),
        (0, 2, 1, 3))
    out5 = _sc_embed(tok4, jnp.asarray(table, jnp.float32))
    return (out5.reshape(SEQ, D // 8, NW, 8, BW)
            .transpose(2, 4, 0, 1, 3)
            .reshape(BATCH, SEQ, D))
